# Initial kernel scaffold; baseline (speedup 1.0000x reference)
import jax
import jax.numpy as jnp
from jax.experimental import pallas as pl


def kernel(x_atom_type, x_degree, x_charge, x_hybridization, edge_index, batch, ptr, emb_atom, emb_deg, emb_chg, emb_hyb, Wb1, bb1, Wb2, bb2, Wg1, bg1, Wg2, bg2, Wg3, bg3, ln_g, ln_b, Wa1, ba1, Wa2, ba2):
    raise NotImplementedError("write your pallas kernel here")


# speedup vs baseline: 4.6276x; 4.6276x over previous
import functools

import jax
import jax.numpy as jnp
from jax import lax
from jax.experimental import pallas as pl
from jax.experimental.pallas import tpu as pltpu
from jax.experimental.pallas import tpu_sc as plsc

F32 = jnp.float32
I32 = jnp.int32

N_PAD = 10240
BN = 512
NT = N_PAD // BN
NSUB = 16
NCORE = 2
ROWS_PER_SUB = N_PAD // NSUB
EG = 128


def _sc_pass(x2, src2, dst16, n_groups):
    mesh = plsc.VectorSubcoreMesh(core_axis_name="c", subcore_axis_name="s")

    @functools.partial(
        pl.kernel,
        out_type=jax.ShapeDtypeStruct((2 * N_PAD, 128), F32),
        mesh=mesh,
        scratch_types=[
            pltpu.VMEM((n_groups, EG), I32),
            pltpu.VMEM((n_groups, EG), I32),
            pltpu.VMEM((EG, 128), F32),
            pltpu.VMEM_SHARED((N_PAD, 128), F32),
        ],
    )
    def k(x_hbm, src_hbm, dst_hbm, h_hbm, src_v, dst_v, rows_v, acc_sh):
        c = lax.axis_index("c")
        s = lax.axis_index("s")
        w = c * NSUB + s
        pltpu.sync_copy(src_hbm.at[w], src_v)
        pltpu.sync_copy(dst_hbm.at[s], dst_v)
        row0 = s * ROWS_PER_SUB
        xoff = c * N_PAD
        pltpu.sync_copy(x_hbm.at[pl.ds(xoff + row0, ROWS_PER_SUB)],
                        acc_sh.at[pl.ds(row0, ROWS_PER_SUB)])
        plsc.subcore_barrier()

        @pl.loop(0, n_groups)
        def _(g):
            pltpu.sync_copy(x_hbm.at[src_v.at[g]], rows_v)
            pltpu.sync_copy(rows_v, acc_sh.at[dst_v.at[g]], add=True)

        plsc.subcore_barrier()
        pltpu.sync_copy(acc_sh.at[pl.ds(row0, ROWS_PER_SUB)],
                        h_hbm.at[pl.ds(xoff + row0, ROWS_PER_SUB)])

    return k(x2, src2, dst16)


def _full(shape):
    return pl.BlockSpec(shape, lambda i: tuple(0 for _ in shape))


def _k1_body(aux_ref, wemb_ref, wb1_ref, bb1_ref, wb2_ref, bb2_ref, out_ref):
    vio = lax.broadcasted_iota(I32, (BN, 128), 1)
    oh = jnp.zeros((BN, 128), F32)
    for k in range(4):
        oh = oh + (aux_ref[:, k:k + 1] == vio).astype(F32)
    t = jnp.dot(oh, wemb_ref[...], preferred_element_type=F32)
    t = jnp.maximum(jnp.dot(t, wb1_ref[...], preferred_element_type=F32)
                    + bb1_ref[...], 0.0)
    x = jnp.dot(t, wb2_ref[...], preferred_element_type=F32) + bb2_ref[...]
    out_ref[0] = x[:, :128]
    out_ref[1] = x[:, 128:]


def _k2_body(h_ref, wg1_ref, bg1_ref, wg2_ref, bg2_ref, wg3_ref, bg3_ref,
             lng_ref, lnb_ref, out_ref):
    h = jnp.concatenate([h_ref[0], h_ref[1]], axis=1)
    h = jnp.maximum(jnp.dot(h, wg1_ref[...], preferred_element_type=F32)
                    + bg1_ref[...], 0.0)
    h = jnp.maximum(jnp.dot(h, wg2_ref[...], preferred_element_type=F32)
                    + bg2_ref[...], 0.0)
    h = jnp.dot(h, wg3_ref[...], preferred_element_type=F32) + bg3_ref[...]
    mu = jnp.mean(h, axis=1, keepdims=True)
    d = h - mu
    var = jnp.mean(d * d, axis=1, keepdims=True)
    xn = lng_ref[...] * (d * lax.rsqrt(var + 1e-5)) + lnb_ref[...]
    out_ref[0] = xn[:, :128]
    out_ref[1] = xn[:, 128:]


def _k3_body(nmol, x1_ref, x2_ref, x3_ref, aux_ref, wa1_ref, ba1_ref,
             wa2_ref, ba2_ref, out_ref):
    i = pl.program_id(0)
    cat = jnp.concatenate([x1_ref[0], x1_ref[1], x2_ref[0], x2_ref[1],
                           x3_ref[0], x3_ref[1]], axis=1)
    y = jnp.maximum(jnp.dot(cat, wa1_ref[...], preferred_element_type=F32)
                    + ba1_ref[...], 0.0)
    z = jnp.dot(y, wa2_ref[...], preferred_element_type=F32) + ba2_ref[...]
    sel = (aux_ref[:, 4:5] == lax.broadcasted_iota(I32, (BN, nmol), 1)).astype(F32)
    contrib = lax.dot_general(sel, z, (((0,), (0,)), ((), ())),
                              preferred_element_type=F32)

    @pl.when(i == 0)
    def _():
        out_ref[...] = jnp.zeros_like(out_ref)

    out_ref[...] += contrib


def kernel(x_atom_type, x_degree, x_charge, x_hybridization, edge_index,
           batch, ptr, emb_atom, emb_deg, emb_chg, emb_hyb, Wb1, bb1, Wb2,
           bb2, Wg1, bg1, Wg2, bg2, Wg3, bg3, ln_g, ln_b, Wa1, ba1, Wa2, ba2):
    n = x_atom_type.shape[0]
    e = edge_index.shape[1]
    nmol = ptr.shape[0] - 1
    emb = emb_atom.shape[1]
    dim = Wb2.shape[1]
    out_dim = Wa2.shape[1]
    pad_n = N_PAD - n

    o1 = emb_atom.shape[0]
    o2 = o1 + emb_deg.shape[0]
    o3 = o2 + emb_chg.shape[0]
    vocab = o3 + emb_hyb.shape[0]
    aux = jnp.zeros((N_PAD, 8), I32)
    aux = aux.at[:n, 0].set(x_atom_type.astype(I32))
    aux = aux.at[:n, 1].set(x_degree.astype(I32) + o1)
    aux = aux.at[:n, 2].set(x_charge.astype(I32) + o2)
    aux = aux.at[:n, 3].set(x_hybridization.astype(I32) + o3)
    aux = aux.at[n:, :4].set(vocab)
    aux = aux.at[:n, 4].set(batch.astype(I32))
    aux = aux.at[n:, 4].set(nmol)

    w_emb = jnp.zeros((128, 4 * emb), F32)
    w_emb = w_emb.at[:o1, :emb].set(emb_atom)
    w_emb = w_emb.at[o1:o2, emb:2 * emb].set(emb_deg)
    w_emb = w_emb.at[o2:o3, 2 * emb:3 * emb].set(emb_chg)
    w_emb = w_emb.at[o3:vocab, 3 * emb:].set(emb_hyb)

    src = edge_index[0].astype(I32)
    dst = edge_index[1].astype(I32)
    e_pad = -(-e // (NSUB * EG)) * (NSUB * EG)
    n_groups = e_pad // (NSUB * EG)
    pad_e = e_pad - e
    pad_ar = jnp.arange(pad_e, dtype=I32)
    src_p = jnp.concatenate([src, pad_ar % n]).reshape(NSUB, n_groups, EG)
    dst_p = jnp.concatenate([dst, n + pad_ar % pad_n]).reshape(NSUB, n_groups, EG)
    src2 = jnp.concatenate([src_p, src_p + N_PAD], axis=0)

    b1 = bb1.reshape(1, -1)
    b2 = bb2.reshape(1, -1)
    g1 = bg1.reshape(1, -1)
    g2 = bg2.reshape(1, -1)
    g3 = bg3.reshape(1, -1)
    a1 = ba1.reshape(1, -1)
    a2 = ba2.reshape(1, -1)
    lng = ln_g.reshape(1, -1)
    lnb = ln_b.reshape(1, -1)

    xspec = pl.BlockSpec((2, BN, 128), lambda i: (0, i, 0))
    xshape = jax.ShapeDtypeStruct((2, N_PAD, 128), F32)

    x = pl.pallas_call(
        _k1_body,
        grid=(NT,),
        in_specs=[
            pl.BlockSpec((BN, 8), lambda i: (i, 0)),
            _full((128, 4 * emb)),
            _full(Wb1.shape), _full((1, Wb1.shape[1])),
            _full(Wb2.shape), _full((1, dim)),
        ],
        out_specs=xspec,
        out_shape=xshape,
    )(aux, w_emb, Wb1, b1, Wb2, b2)

    gin = pl.pallas_call(
        _k2_body,
        grid=(NT,),
        in_specs=[
            xspec,
            _full(Wg1.shape), _full((1, dim)),
            _full(Wg2.shape), _full((1, dim)),
            _full(Wg3.shape), _full((1, dim)),
            _full((1, dim)), _full((1, dim)),
        ],
        out_specs=xspec,
        out_shape=xshape,
    )
    outs = []
    for _ in range(3):
        h2 = _sc_pass(x.reshape(2 * N_PAD, 128), src2, dst_p, n_groups)
        x = gin(h2.reshape(2, N_PAD, 128), Wg1, g1, Wg2, g2, Wg3, g3, lng, lnb)
        outs.append(x)

    out = pl.pallas_call(
        functools.partial(_k3_body, nmol),
        grid=(NT,),
        in_specs=[
            xspec, xspec, xspec,
            pl.BlockSpec((BN, 8), lambda i: (i, 0)),
            _full(Wa1.shape), _full((1, Wa1.shape[1])),
            _full(Wa2.shape), _full((1, out_dim)),
        ],
        out_specs=pl.BlockSpec((nmol, out_dim), lambda i: (0, 0)),
        out_shape=jax.ShapeDtypeStruct((nmol, out_dim), F32),
    )(outs[0], outs[1], outs[2], aux, Wa1, a1, Wa2, a2)
    return out


# speedup vs baseline: 6.0670x; 1.3110x over previous
import functools

import jax
import jax.numpy as jnp
from jax import lax
from jax.experimental import pallas as pl
from jax.experimental.pallas import tpu as pltpu
from jax.experimental.pallas import tpu_sc as plsc

F32 = jnp.float32
I32 = jnp.int32

N_PAD = 10240
BN = 512
NT = N_PAD // BN
NSUB = 16
NCORE = 2
ROWS_PER_SUB = N_PAD // NSUB
EG = 128


def _sc_pass(x2, src2, dst16, n_groups):
    mesh = plsc.VectorSubcoreMesh(core_axis_name="c", subcore_axis_name="s")

    nchunk = 2
    cg = n_groups // nchunk

    @functools.partial(
        pl.kernel,
        out_type=jax.ShapeDtypeStruct((2 * N_PAD, 128), F32),
        mesh=mesh,
        scratch_types=[
            pltpu.VMEM((cg, EG), I32),
            pltpu.VMEM((cg, EG), I32),
            pltpu.VMEM((EG, 128), F32),
            pltpu.VMEM((EG, 128), F32),
            pltpu.VMEM_SHARED((N_PAD, 128), F32),
            pltpu.SemaphoreType.DMA,
            pltpu.SemaphoreType.DMA,
        ],
    )
    def k(x_hbm, src_hbm, dst_hbm, h_hbm, src_v, dst_v, rows_a, rows_b,
          acc_sh, sem_a, sem_b):
        c = lax.axis_index("c")
        s = lax.axis_index("s")
        w = c * NSUB + s
        row0 = s * ROWS_PER_SUB
        xoff = c * N_PAD
        pltpu.sync_copy(x_hbm.at[pl.ds(xoff + row0, ROWS_PER_SUB)],
                        acc_sh.at[pl.ds(row0, ROWS_PER_SUB)])
        plsc.subcore_barrier()

        for chunk in range(nchunk):
            pltpu.sync_copy(src_hbm.at[w].at[chunk], src_v)
            pltpu.sync_copy(dst_hbm.at[s].at[chunk], dst_v)
            pltpu.async_copy(x_hbm.at[src_v.at[0]], rows_a, sem_a)

            @pl.loop(0, cg, step=2)
            def _(g):
                pltpu.async_copy(x_hbm.at[src_v.at[g + 1]], rows_b, sem_b)
                pltpu.make_async_copy(x_hbm.at[src_v.at[g]], rows_a, sem_a).wait()
                pltpu.sync_copy(rows_a, acc_sh.at[dst_v.at[g]], add=True)

                @pl.when(g + 2 < cg)
                def _():
                    pltpu.async_copy(x_hbm.at[src_v.at[g + 2]], rows_a, sem_a)

                pltpu.make_async_copy(x_hbm.at[src_v.at[g + 1]], rows_b, sem_b).wait()
                pltpu.sync_copy(rows_b, acc_sh.at[dst_v.at[g + 1]], add=True)

        plsc.subcore_barrier()
        pltpu.sync_copy(acc_sh.at[pl.ds(row0, ROWS_PER_SUB)],
                        h_hbm.at[pl.ds(xoff + row0, ROWS_PER_SUB)])

    return k(x2, src2, dst16)


def _full(shape):
    return pl.BlockSpec(shape, lambda i: tuple(0 for _ in shape))


def _k1_body(aux_ref, wemb_ref, wb1_ref, bb1_ref, wb2_ref, bb2_ref, out_ref):
    vio = lax.broadcasted_iota(I32, (BN, 128), 1)
    oh = jnp.zeros((BN, 128), F32)
    for k in range(4):
        oh = oh + (aux_ref[:, k:k + 1] == vio).astype(F32)
    t = jnp.dot(oh, wemb_ref[...], preferred_element_type=F32)
    t = jnp.maximum(jnp.dot(t, wb1_ref[...], preferred_element_type=F32)
                    + bb1_ref[...], 0.0)
    x = jnp.dot(t, wb2_ref[...], preferred_element_type=F32) + bb2_ref[...]
    out_ref[0] = x[:, :128]
    out_ref[1] = x[:, 128:]


def _k2_body(h_ref, wg1_ref, bg1_ref, wg2_ref, bg2_ref, wg3_ref, bg3_ref,
             lng_ref, lnb_ref, out_ref):
    h = jnp.concatenate([h_ref[0], h_ref[1]], axis=1)
    h = jnp.maximum(jnp.dot(h, wg1_ref[...], preferred_element_type=F32)
                    + bg1_ref[...], 0.0)
    h = jnp.maximum(jnp.dot(h, wg2_ref[...], preferred_element_type=F32)
                    + bg2_ref[...], 0.0)
    h = jnp.dot(h, wg3_ref[...], preferred_element_type=F32) + bg3_ref[...]
    mu = jnp.mean(h, axis=1, keepdims=True)
    d = h - mu
    var = jnp.mean(d * d, axis=1, keepdims=True)
    xn = lng_ref[...] * (d * lax.rsqrt(var + 1e-5)) + lnb_ref[...]
    out_ref[0] = xn[:, :128]
    out_ref[1] = xn[:, 128:]


def _k3_body(nmol, x1_ref, x2_ref, x3_ref, aux_ref, wa1_ref, ba1_ref,
             wa2_ref, ba2_ref, out_ref):
    i = pl.program_id(0)
    cat = jnp.concatenate([x1_ref[0], x1_ref[1], x2_ref[0], x2_ref[1],
                           x3_ref[0], x3_ref[1]], axis=1)
    y = jnp.maximum(jnp.dot(cat, wa1_ref[...], preferred_element_type=F32)
                    + ba1_ref[...], 0.0)
    z = jnp.dot(y, wa2_ref[...], preferred_element_type=F32) + ba2_ref[...]
    sel = (aux_ref[:, 4:5] == lax.broadcasted_iota(I32, (BN, nmol), 1)).astype(F32)
    contrib = lax.dot_general(sel, z, (((0,), (0,)), ((), ())),
                              preferred_element_type=F32)

    @pl.when(i == 0)
    def _():
        out_ref[...] = jnp.zeros_like(out_ref)

    out_ref[...] += contrib


def kernel(x_atom_type, x_degree, x_charge, x_hybridization, edge_index,
           batch, ptr, emb_atom, emb_deg, emb_chg, emb_hyb, Wb1, bb1, Wb2,
           bb2, Wg1, bg1, Wg2, bg2, Wg3, bg3, ln_g, ln_b, Wa1, ba1, Wa2, ba2):
    n = x_atom_type.shape[0]
    e = edge_index.shape[1]
    nmol = ptr.shape[0] - 1
    emb = emb_atom.shape[1]
    dim = Wb2.shape[1]
    out_dim = Wa2.shape[1]
    pad_n = N_PAD - n

    o1 = emb_atom.shape[0]
    o2 = o1 + emb_deg.shape[0]
    o3 = o2 + emb_chg.shape[0]
    vocab = o3 + emb_hyb.shape[0]
    aux = jnp.zeros((N_PAD, 8), I32)
    aux = aux.at[:n, 0].set(x_atom_type.astype(I32))
    aux = aux.at[:n, 1].set(x_degree.astype(I32) + o1)
    aux = aux.at[:n, 2].set(x_charge.astype(I32) + o2)
    aux = aux.at[:n, 3].set(x_hybridization.astype(I32) + o3)
    aux = aux.at[n:, :4].set(vocab)
    aux = aux.at[:n, 4].set(batch.astype(I32))
    aux = aux.at[n:, 4].set(nmol)

    w_emb = jnp.zeros((128, 4 * emb), F32)
    w_emb = w_emb.at[:o1, :emb].set(emb_atom)
    w_emb = w_emb.at[o1:o2, emb:2 * emb].set(emb_deg)
    w_emb = w_emb.at[o2:o3, 2 * emb:3 * emb].set(emb_chg)
    w_emb = w_emb.at[o3:vocab, 3 * emb:].set(emb_hyb)

    src = edge_index[0].astype(I32)
    dst = edge_index[1].astype(I32)
    e_pad = -(-e // (NSUB * EG * 4)) * (NSUB * EG * 4)
    n_groups = e_pad // (NSUB * EG)
    pad_e = e_pad - e
    pad_ar = jnp.arange(pad_e, dtype=I32)
    n_chunk_g = n_groups // 2
    src_p = jnp.concatenate([src, pad_ar % n]).reshape(NSUB, 2, n_chunk_g, EG)
    dst_p = jnp.concatenate([dst, n + pad_ar % pad_n]).reshape(NSUB, 2, n_chunk_g, EG)
    src2 = jnp.concatenate([src_p, src_p + N_PAD], axis=0)

    b1 = bb1.reshape(1, -1)
    b2 = bb2.reshape(1, -1)
    g1 = bg1.reshape(1, -1)
    g2 = bg2.reshape(1, -1)
    g3 = bg3.reshape(1, -1)
    a1 = ba1.reshape(1, -1)
    a2 = ba2.reshape(1, -1)
    lng = ln_g.reshape(1, -1)
    lnb = ln_b.reshape(1, -1)

    xspec = pl.BlockSpec((2, BN, 128), lambda i: (0, i, 0))
    xshape = jax.ShapeDtypeStruct((2, N_PAD, 128), F32)

    x = pl.pallas_call(
        _k1_body,
        grid=(NT,),
        in_specs=[
            pl.BlockSpec((BN, 8), lambda i: (i, 0)),
            _full((128, 4 * emb)),
            _full(Wb1.shape), _full((1, Wb1.shape[1])),
            _full(Wb2.shape), _full((1, dim)),
        ],
        out_specs=xspec,
        out_shape=xshape,
    )(aux, w_emb, Wb1, b1, Wb2, b2)

    gin = pl.pallas_call(
        _k2_body,
        grid=(NT,),
        in_specs=[
            xspec,
            _full(Wg1.shape), _full((1, dim)),
            _full(Wg2.shape), _full((1, dim)),
            _full(Wg3.shape), _full((1, dim)),
            _full((1, dim)), _full((1, dim)),
        ],
        out_specs=xspec,
        out_shape=xshape,
    )
    outs = []
    for _ in range(3):
        h2 = _sc_pass(x.reshape(2 * N_PAD, 128), src2, dst_p, n_groups)
        x = gin(h2.reshape(2, N_PAD, 128), Wg1, g1, Wg2, g2, Wg3, g3, lng, lnb)
        outs.append(x)

    out = pl.pallas_call(
        functools.partial(_k3_body, nmol),
        grid=(NT,),
        in_specs=[
            xspec, xspec, xspec,
            pl.BlockSpec((BN, 8), lambda i: (i, 0)),
            _full(Wa1.shape), _full((1, Wa1.shape[1])),
            _full(Wa2.shape), _full((1, out_dim)),
        ],
        out_specs=pl.BlockSpec((nmol, out_dim), lambda i: (0, 0)),
        out_shape=jax.ShapeDtypeStruct((nmol, out_dim), F32),
    )(outs[0], outs[1], outs[2], aux, Wa1, a1, Wa2, a2)
    return out


# speedup vs baseline: 6.0902x; 1.0038x over previous
import functools

import jax
import jax.numpy as jnp
from jax import lax
from jax.experimental import pallas as pl
from jax.experimental.pallas import tpu as pltpu
from jax.experimental.pallas import tpu_sc as plsc

F32 = jnp.float32
BF16 = jnp.bfloat16
I32 = jnp.int32


def _mm(a, b_ref):
    return jnp.dot(a.astype(BF16), b_ref[...], preferred_element_type=F32)

N_PAD = 10240
BN = 512
NT = N_PAD // BN
NSUB = 16
NCORE = 2
ROWS_PER_SUB = N_PAD // NSUB
EG = 128


def _sc_pass(x2, src2, dst16, n_groups):
    mesh = plsc.VectorSubcoreMesh(core_axis_name="c", subcore_axis_name="s")

    nchunk = 2
    cg = n_groups // nchunk

    @functools.partial(
        pl.kernel,
        out_type=jax.ShapeDtypeStruct((2 * N_PAD, 128), F32),
        mesh=mesh,
        scratch_types=[
            pltpu.VMEM((cg, EG), I32),
            pltpu.VMEM((cg, EG), I32),
            pltpu.VMEM((EG, 128), F32),
            pltpu.VMEM((EG, 128), F32),
            pltpu.VMEM_SHARED((N_PAD, 128), F32),
            pltpu.SemaphoreType.DMA,
            pltpu.SemaphoreType.DMA,
        ],
    )
    def k(x_hbm, src_hbm, dst_hbm, h_hbm, src_v, dst_v, rows_a, rows_b,
          acc_sh, sem_a, sem_b):
        c = lax.axis_index("c")
        s = lax.axis_index("s")
        w = c * NSUB + s
        row0 = s * ROWS_PER_SUB
        xoff = c * N_PAD
        pltpu.sync_copy(x_hbm.at[pl.ds(xoff + row0, ROWS_PER_SUB)],
                        acc_sh.at[pl.ds(row0, ROWS_PER_SUB)])
        plsc.subcore_barrier()

        for chunk in range(nchunk):
            pltpu.sync_copy(src_hbm.at[w].at[chunk], src_v)
            pltpu.sync_copy(dst_hbm.at[s].at[chunk], dst_v)
            pltpu.async_copy(x_hbm.at[src_v.at[0]], rows_a, sem_a)

            @pl.loop(0, cg, step=2)
            def _(g):
                pltpu.async_copy(x_hbm.at[src_v.at[g + 1]], rows_b, sem_b)
                pltpu.make_async_copy(x_hbm.at[src_v.at[g]], rows_a, sem_a).wait()
                pltpu.sync_copy(rows_a, acc_sh.at[dst_v.at[g]], add=True)

                @pl.when(g + 2 < cg)
                def _():
                    pltpu.async_copy(x_hbm.at[src_v.at[g + 2]], rows_a, sem_a)

                pltpu.make_async_copy(x_hbm.at[src_v.at[g + 1]], rows_b, sem_b).wait()
                pltpu.sync_copy(rows_b, acc_sh.at[dst_v.at[g + 1]], add=True)

        plsc.subcore_barrier()
        pltpu.sync_copy(acc_sh.at[pl.ds(row0, ROWS_PER_SUB)],
                        h_hbm.at[pl.ds(xoff + row0, ROWS_PER_SUB)])

    return k(x2, src2, dst16)


def _full(shape):
    return pl.BlockSpec(shape, lambda i: tuple(0 for _ in shape))


def _k1_body(aux_ref, wemb_ref, wb1_ref, bb1_ref, wb2_ref, bb2_ref, out_ref):
    vio = lax.broadcasted_iota(I32, (BN, 128), 1)
    oh = jnp.zeros((BN, 128), BF16)
    for k in range(4):
        oh = oh + (aux_ref[:, k:k + 1] == vio).astype(BF16)
    t = jnp.dot(oh, wemb_ref[...], preferred_element_type=F32)
    t = jnp.maximum(_mm(t, wb1_ref) + bb1_ref[...], 0.0)
    x = _mm(t, wb2_ref) + bb2_ref[...]
    out_ref[0] = x[:, :128]
    out_ref[1] = x[:, 128:]


def _k2_body(h_ref, wg1_ref, bg1_ref, wg2_ref, bg2_ref, wg3_ref, bg3_ref,
             lng_ref, lnb_ref, out_ref):
    h = jnp.concatenate([h_ref[0], h_ref[1]], axis=1)
    h = jnp.maximum(_mm(h, wg1_ref) + bg1_ref[...], 0.0)
    h = jnp.maximum(_mm(h, wg2_ref) + bg2_ref[...], 0.0)
    h = _mm(h, wg3_ref) + bg3_ref[...]
    mu = jnp.mean(h, axis=1, keepdims=True)
    d = h - mu
    var = jnp.mean(d * d, axis=1, keepdims=True)
    xn = lng_ref[...] * (d * lax.rsqrt(var + 1e-5)) + lnb_ref[...]
    out_ref[0] = xn[:, :128]
    out_ref[1] = xn[:, 128:]


def _k3_body(nmol, x1_ref, x2_ref, x3_ref, aux_ref, wa1_ref, ba1_ref,
             wa2_ref, ba2_ref, out_ref):
    i = pl.program_id(0)
    cat = jnp.concatenate([x1_ref[0], x1_ref[1], x2_ref[0], x2_ref[1],
                           x3_ref[0], x3_ref[1]], axis=1)
    y = jnp.maximum(_mm(cat, wa1_ref) + ba1_ref[...], 0.0)
    z = _mm(y, wa2_ref) + ba2_ref[...]
    sel = (aux_ref[:, 4:5] == lax.broadcasted_iota(I32, (BN, nmol), 1)).astype(BF16)
    contrib = lax.dot_general(sel, z.astype(BF16), (((0,), (0,)), ((), ())),
                              preferred_element_type=F32)

    @pl.when(i == 0)
    def _():
        out_ref[...] = jnp.zeros_like(out_ref)

    out_ref[...] += contrib


def kernel(x_atom_type, x_degree, x_charge, x_hybridization, edge_index,
           batch, ptr, emb_atom, emb_deg, emb_chg, emb_hyb, Wb1, bb1, Wb2,
           bb2, Wg1, bg1, Wg2, bg2, Wg3, bg3, ln_g, ln_b, Wa1, ba1, Wa2, ba2):
    n = x_atom_type.shape[0]
    e = edge_index.shape[1]
    nmol = ptr.shape[0] - 1
    emb = emb_atom.shape[1]
    dim = Wb2.shape[1]
    out_dim = Wa2.shape[1]
    pad_n = N_PAD - n

    o1 = emb_atom.shape[0]
    o2 = o1 + emb_deg.shape[0]
    o3 = o2 + emb_chg.shape[0]
    vocab = o3 + emb_hyb.shape[0]
    aux = jnp.zeros((N_PAD, 8), I32)
    aux = aux.at[:n, 0].set(x_atom_type.astype(I32))
    aux = aux.at[:n, 1].set(x_degree.astype(I32) + o1)
    aux = aux.at[:n, 2].set(x_charge.astype(I32) + o2)
    aux = aux.at[:n, 3].set(x_hybridization.astype(I32) + o3)
    aux = aux.at[n:, :4].set(vocab)
    aux = aux.at[:n, 4].set(batch.astype(I32))
    aux = aux.at[n:, 4].set(nmol)

    w_emb = jnp.zeros((128, 4 * emb), F32)
    w_emb = w_emb.at[:o1, :emb].set(emb_atom)
    w_emb = w_emb.at[o1:o2, emb:2 * emb].set(emb_deg)
    w_emb = w_emb.at[o2:o3, 2 * emb:3 * emb].set(emb_chg)
    w_emb = w_emb.at[o3:vocab, 3 * emb:].set(emb_hyb)

    src = edge_index[0].astype(I32)
    dst = edge_index[1].astype(I32)
    e_pad = -(-e // (NSUB * EG * 4)) * (NSUB * EG * 4)
    n_groups = e_pad // (NSUB * EG)
    pad_e = e_pad - e
    pad_ar = jnp.arange(pad_e, dtype=I32)
    n_chunk_g = n_groups // 2
    src_p = jnp.concatenate([src, pad_ar % n]).reshape(NSUB, 2, n_chunk_g, EG)
    dst_p = jnp.concatenate([dst, n + pad_ar % pad_n]).reshape(NSUB, 2, n_chunk_g, EG)
    src2 = jnp.concatenate([src_p, src_p + N_PAD], axis=0)

    b1 = bb1.reshape(1, -1)
    b2 = bb2.reshape(1, -1)
    g1 = bg1.reshape(1, -1)
    g2 = bg2.reshape(1, -1)
    g3 = bg3.reshape(1, -1)
    a1 = ba1.reshape(1, -1)
    a2 = ba2.reshape(1, -1)
    lng = ln_g.reshape(1, -1)
    lnb = ln_b.reshape(1, -1)

    xspec = pl.BlockSpec((2, BN, 128), lambda i: (0, i, 0))
    xshape = jax.ShapeDtypeStruct((2, N_PAD, 128), F32)

    x = pl.pallas_call(
        _k1_body,
        grid=(NT,),
        in_specs=[
            pl.BlockSpec((BN, 8), lambda i: (i, 0)),
            _full((128, 4 * emb)),
            _full(Wb1.shape), _full((1, Wb1.shape[1])),
            _full(Wb2.shape), _full((1, dim)),
        ],
        out_specs=xspec,
        out_shape=xshape,
    )(aux, w_emb.astype(BF16), Wb1.astype(BF16), b1, Wb2.astype(BF16), b2)

    gin = pl.pallas_call(
        _k2_body,
        grid=(NT,),
        in_specs=[
            xspec,
            _full(Wg1.shape), _full((1, dim)),
            _full(Wg2.shape), _full((1, dim)),
            _full(Wg3.shape), _full((1, dim)),
            _full((1, dim)), _full((1, dim)),
        ],
        out_specs=xspec,
        out_shape=xshape,
    )
    Wg1b, Wg2b, Wg3b = Wg1.astype(BF16), Wg2.astype(BF16), Wg3.astype(BF16)
    outs = []
    for _ in range(3):
        h2 = _sc_pass(x.reshape(2 * N_PAD, 128), src2, dst_p, n_groups)
        x = gin(h2.reshape(2, N_PAD, 128), Wg1b, g1, Wg2b, g2, Wg3b, g3, lng, lnb)
        outs.append(x)

    out = pl.pallas_call(
        functools.partial(_k3_body, nmol),
        grid=(NT,),
        in_specs=[
            xspec, xspec, xspec,
            pl.BlockSpec((BN, 8), lambda i: (i, 0)),
            _full(Wa1.shape), _full((1, Wa1.shape[1])),
            _full(Wa2.shape), _full((1, out_dim)),
        ],
        out_specs=pl.BlockSpec((nmol, out_dim), lambda i: (0, 0)),
        out_shape=jax.ShapeDtypeStruct((nmol, out_dim), F32),
    )(outs[0], outs[1], outs[2], aux, Wa1.astype(BF16), a1, Wa2.astype(BF16), a2)
    return out


# speedup vs baseline: 7.7368x; 1.2704x over previous
import functools

import jax
import jax.numpy as jnp
from jax import lax
from jax.experimental import pallas as pl
from jax.experimental.pallas import tpu as pltpu
from jax.experimental.pallas import tpu_sc as plsc

F32 = jnp.float32
BF16 = jnp.bfloat16
I32 = jnp.int32


def _mm(a, b_ref):
    return jnp.dot(a.astype(BF16), b_ref[...], preferred_element_type=F32)

N_PAD = 10240
BN = 512
NT = N_PAD // BN
NSUB = 16
NCORE = 2
ROWS_PER_SUB = N_PAD // NSUB
EG = 128


def _sc_pass(x2, src2, dst16, n_groups):
    mesh = plsc.VectorSubcoreMesh(core_axis_name="c", subcore_axis_name="s")

    nchunk = 2
    cg = n_groups // nchunk

    @functools.partial(
        pl.kernel,
        out_type=jax.ShapeDtypeStruct((2 * N_PAD, 128), F32),
        mesh=mesh,
        scratch_types=[
            pltpu.VMEM((cg, EG), I32),
            pltpu.VMEM((cg, EG), I32),
            pltpu.VMEM((EG, 128), F32),
            pltpu.VMEM((EG, 128), F32),
            pltpu.VMEM_SHARED((N_PAD, 128), F32),
            pltpu.SemaphoreType.DMA,
            pltpu.SemaphoreType.DMA,
        ],
    )
    def k(x_hbm, src_hbm, dst_hbm, h_hbm, src_v, dst_v, rows_a, rows_b,
          acc_sh, sem_a, sem_b):
        c = lax.axis_index("c")
        s = lax.axis_index("s")
        w = c * NSUB + s
        row0 = s * ROWS_PER_SUB
        xoff = c * N_PAD
        pltpu.sync_copy(x_hbm.at[pl.ds(xoff + row0, ROWS_PER_SUB)],
                        acc_sh.at[pl.ds(row0, ROWS_PER_SUB)])
        plsc.subcore_barrier()

        for chunk in range(nchunk):
            pltpu.sync_copy(src_hbm.at[w].at[chunk], src_v)
            pltpu.sync_copy(dst_hbm.at[s].at[chunk], dst_v)
            pltpu.async_copy(x_hbm.at[src_v.at[0]], rows_a, sem_a)

            @pl.loop(0, cg, step=2)
            def _(g):
                pltpu.async_copy(x_hbm.at[src_v.at[g + 1]], rows_b, sem_b)
                pltpu.make_async_copy(x_hbm.at[src_v.at[g]], rows_a, sem_a).wait()
                pltpu.sync_copy(rows_a, acc_sh.at[dst_v.at[g]], add=True)

                @pl.when(g + 2 < cg)
                def _():
                    pltpu.async_copy(x_hbm.at[src_v.at[g + 2]], rows_a, sem_a)

                pltpu.make_async_copy(x_hbm.at[src_v.at[g + 1]], rows_b, sem_b).wait()
                pltpu.sync_copy(rows_b, acc_sh.at[dst_v.at[g + 1]], add=True)

        plsc.subcore_barrier()
        pltpu.sync_copy(acc_sh.at[pl.ds(row0, ROWS_PER_SUB)],
                        h_hbm.at[pl.ds(xoff + row0, ROWS_PER_SUB)])

    return k(x2, src2, dst16)


def _full(shape):
    return pl.BlockSpec(shape, lambda i: tuple(0 for _ in shape))


def _k1_body(aux_ref, wemb_ref, wb1_ref, bb1_ref, wb2_ref, bb2_ref, out_ref):
    vio = lax.broadcasted_iota(I32, (BN, 128), 1)
    oh = jnp.zeros((BN, 128), BF16)
    for k in range(4):
        oh = oh + (aux_ref[:, k:k + 1] == vio).astype(BF16)
    t = jnp.dot(oh, wemb_ref[...], preferred_element_type=F32)
    t = jnp.maximum(_mm(t, wb1_ref) + bb1_ref[...], 0.0)
    x = _mm(t, wb2_ref) + bb2_ref[...]
    out_ref[0] = x[:, :128]
    out_ref[1] = x[:, 128:]


def _k2_body(h_ref, wg1_ref, bg1_ref, wg2_ref, bg2_ref, wg3_ref, bg3_ref,
             lng_ref, lnb_ref, out_ref):
    h = jnp.concatenate([h_ref[0], h_ref[1]], axis=1)
    h = jnp.maximum(_mm(h, wg1_ref) + bg1_ref[...], 0.0)
    h = jnp.maximum(_mm(h, wg2_ref) + bg2_ref[...], 0.0)
    h = _mm(h, wg3_ref) + bg3_ref[...]
    mu = jnp.mean(h, axis=1, keepdims=True)
    d = h - mu
    var = jnp.mean(d * d, axis=1, keepdims=True)
    xn = lng_ref[...] * (d * lax.rsqrt(var + 1e-5)) + lnb_ref[...]
    out_ref[0] = xn[:, :128]
    out_ref[1] = xn[:, 128:]


def _k3_body(nmol, x1_ref, x2_ref, x3_ref, aux_ref, wa1_ref, ba1_ref,
             wa2_ref, ba2_ref, out_ref):
    i = pl.program_id(0)
    cat = jnp.concatenate([x1_ref[0], x1_ref[1], x2_ref[0], x2_ref[1],
                           x3_ref[0], x3_ref[1]], axis=1)
    y = jnp.maximum(_mm(cat, wa1_ref) + ba1_ref[...], 0.0)
    z = _mm(y, wa2_ref) + ba2_ref[...]
    sel = (aux_ref[:, 4:5] == lax.broadcasted_iota(I32, (BN, nmol), 1)).astype(BF16)
    contrib = lax.dot_general(sel, z.astype(BF16), (((0,), (0,)), ((), ())),
                              preferred_element_type=F32)

    @pl.when(i == 0)
    def _():
        out_ref[...] = jnp.zeros_like(out_ref)

    out_ref[...] += contrib


def kernel(x_atom_type, x_degree, x_charge, x_hybridization, edge_index,
           batch, ptr, emb_atom, emb_deg, emb_chg, emb_hyb, Wb1, bb1, Wb2,
           bb2, Wg1, bg1, Wg2, bg2, Wg3, bg3, ln_g, ln_b, Wa1, ba1, Wa2, ba2):
    n = x_atom_type.shape[0]
    e = edge_index.shape[1]
    nmol = ptr.shape[0] - 1
    emb = emb_atom.shape[1]
    dim = Wb2.shape[1]
    out_dim = Wa2.shape[1]
    pad_n = N_PAD - n

    o1 = emb_atom.shape[0]
    o2 = o1 + emb_deg.shape[0]
    o3 = o2 + emb_chg.shape[0]
    vocab = o3 + emb_hyb.shape[0]
    zcol = jnp.zeros((n,), I32)
    cols = jnp.stack([x_atom_type.astype(I32), x_degree.astype(I32) + o1,
                      x_charge.astype(I32) + o2,
                      x_hybridization.astype(I32) + o3,
                      batch.astype(I32), zcol, zcol, zcol], axis=1)
    tail = jnp.broadcast_to(
        jnp.array([vocab, vocab, vocab, vocab, nmol, 0, 0, 0], I32),
        (pad_n, 8))
    aux = jnp.concatenate([cols, tail], axis=0)

    w_emb = jnp.concatenate([
        jnp.pad(emb_atom, ((0, 0), (0, 3 * emb))),
        jnp.pad(emb_deg, ((0, 0), (emb, 2 * emb))),
        jnp.pad(emb_chg, ((0, 0), (2 * emb, emb))),
        jnp.pad(emb_hyb, ((0, 0), (3 * emb, 0))),
        jnp.zeros((128 - vocab, 4 * emb), F32)], axis=0)

    src = edge_index[0].astype(I32)
    dst = edge_index[1].astype(I32)
    e_pad = -(-e // (NSUB * EG * 4)) * (NSUB * EG * 4)
    n_groups = e_pad // (NSUB * EG)
    pad_e = e_pad - e
    pad_ar = jnp.arange(pad_e, dtype=I32)
    n_chunk_g = n_groups // 2
    src_g = jnp.concatenate([src, pad_ar % n]).reshape(1, NSUB, 2, n_chunk_g, EG)
    dst_p = jnp.concatenate([dst, n + pad_ar % pad_n]).reshape(NSUB, 2, n_chunk_g, EG)
    src2 = jnp.concatenate([src_g, src_g + N_PAD],
                           axis=0).reshape(2 * NSUB, 2, n_chunk_g, EG)

    b1 = bb1.reshape(1, -1)
    b2 = bb2.reshape(1, -1)
    g1 = bg1.reshape(1, -1)
    g2 = bg2.reshape(1, -1)
    g3 = bg3.reshape(1, -1)
    a1 = ba1.reshape(1, -1)
    a2 = ba2.reshape(1, -1)
    lng = ln_g.reshape(1, -1)
    lnb = ln_b.reshape(1, -1)

    xspec = pl.BlockSpec((2, BN, 128), lambda i: (0, i, 0))
    xshape = jax.ShapeDtypeStruct((2, N_PAD, 128), F32)

    x = pl.pallas_call(
        _k1_body,
        grid=(NT,),
        in_specs=[
            pl.BlockSpec((BN, 8), lambda i: (i, 0)),
            _full((128, 4 * emb)),
            _full(Wb1.shape), _full((1, Wb1.shape[1])),
            _full(Wb2.shape), _full((1, dim)),
        ],
        out_specs=xspec,
        out_shape=xshape,
    )(aux, w_emb.astype(BF16), Wb1.astype(BF16), b1, Wb2.astype(BF16), b2)

    gin = pl.pallas_call(
        _k2_body,
        grid=(NT,),
        in_specs=[
            xspec,
            _full(Wg1.shape), _full((1, dim)),
            _full(Wg2.shape), _full((1, dim)),
            _full(Wg3.shape), _full((1, dim)),
            _full((1, dim)), _full((1, dim)),
        ],
        out_specs=xspec,
        out_shape=xshape,
    )
    Wg1b, Wg2b, Wg3b = Wg1.astype(BF16), Wg2.astype(BF16), Wg3.astype(BF16)
    outs = []
    for _ in range(3):
        h2 = _sc_pass(x.reshape(2 * N_PAD, 128), src2, dst_p, n_groups)
        x = gin(h2.reshape(2, N_PAD, 128), Wg1b, g1, Wg2b, g2, Wg3b, g3, lng, lnb)
        outs.append(x)

    out = pl.pallas_call(
        functools.partial(_k3_body, nmol),
        grid=(NT,),
        in_specs=[
            xspec, xspec, xspec,
            pl.BlockSpec((BN, 8), lambda i: (i, 0)),
            _full(Wa1.shape), _full((1, Wa1.shape[1])),
            _full(Wa2.shape), _full((1, out_dim)),
        ],
        out_specs=pl.BlockSpec((nmol, out_dim), lambda i: (0, 0)),
        out_shape=jax.ShapeDtypeStruct((nmol, out_dim), F32),
    )(outs[0], outs[1], outs[2], aux, Wa1.astype(BF16), a1, Wa2.astype(BF16), a2)
    return out


# speedup vs baseline: 8.4124x; 1.0873x over previous
import functools

import jax
import jax.numpy as jnp
from jax import lax
from jax.experimental import pallas as pl
from jax.experimental.pallas import tpu as pltpu
from jax.experimental.pallas import tpu_sc as plsc

F32 = jnp.float32
BF16 = jnp.bfloat16
I32 = jnp.int32


def _mm(a, b_ref):
    return jnp.dot(a.astype(BF16), b_ref[...], preferred_element_type=F32)

N_PAD = 10240
BN = 1024
NT = N_PAD // BN
NSUB = 16
NCORE = 2
ROWS_PER_SUB = N_PAD // NSUB
EG = 128


def _sc_pass(x2, src2, dst16, n_groups):
    mesh = plsc.VectorSubcoreMesh(core_axis_name="c", subcore_axis_name="s")

    nchunk = 2
    cg = n_groups // nchunk

    @functools.partial(
        pl.kernel,
        out_type=jax.ShapeDtypeStruct((2 * N_PAD, 128), F32),
        mesh=mesh,
        scratch_types=[
            pltpu.VMEM((cg, EG), I32),
            pltpu.VMEM((cg, EG), I32),
            pltpu.VMEM((EG, 128), F32),
            pltpu.VMEM((EG, 128), F32),
            pltpu.VMEM_SHARED((N_PAD, 128), F32),
            pltpu.SemaphoreType.DMA,
            pltpu.SemaphoreType.DMA,
        ],
    )
    def k(x_hbm, src_hbm, dst_hbm, h_hbm, src_v, dst_v, rows_a, rows_b,
          acc_sh, sem_a, sem_b):
        c = lax.axis_index("c")
        s = lax.axis_index("s")
        row0 = s * ROWS_PER_SUB
        xoff = c * N_PAD
        pltpu.sync_copy(x_hbm.at[pl.ds(xoff + row0, ROWS_PER_SUB)],
                        acc_sh.at[pl.ds(row0, ROWS_PER_SUB)])
        plsc.subcore_barrier()

        for chunk in range(nchunk):
            gbase = s * n_groups + chunk * cg
            pltpu.sync_copy(src_hbm.at[pl.ds(c * (NSUB * n_groups) + gbase, cg)],
                            src_v)
            pltpu.sync_copy(dst_hbm.at[pl.ds(gbase, cg)], dst_v)
            pltpu.async_copy(x_hbm.at[src_v.at[0]], rows_a, sem_a)

            @pl.loop(0, cg, step=2)
            def _(g):
                pltpu.async_copy(x_hbm.at[src_v.at[g + 1]], rows_b, sem_b)
                pltpu.make_async_copy(x_hbm.at[src_v.at[g]], rows_a, sem_a).wait()
                pltpu.sync_copy(rows_a, acc_sh.at[dst_v.at[g]], add=True)

                @pl.when(g + 2 < cg)
                def _():
                    pltpu.async_copy(x_hbm.at[src_v.at[g + 2]], rows_a, sem_a)

                pltpu.make_async_copy(x_hbm.at[src_v.at[g + 1]], rows_b, sem_b).wait()
                pltpu.sync_copy(rows_b, acc_sh.at[dst_v.at[g + 1]], add=True)

        plsc.subcore_barrier()
        pltpu.sync_copy(acc_sh.at[pl.ds(row0, ROWS_PER_SUB)],
                        h_hbm.at[pl.ds(xoff + row0, ROWS_PER_SUB)])

    return k(x2, src2, dst16)


def _full(shape):
    return pl.BlockSpec(shape, lambda i: tuple(0 for _ in shape))


def _k1_body(aux_ref, wemb_ref, wb1_ref, bb1_ref, wb2_ref, bb2_ref, out_ref):
    vio = lax.broadcasted_iota(I32, (BN, 128), 1)
    oh = jnp.zeros((BN, 128), BF16)
    for k in range(4):
        oh = oh + (aux_ref[:, k:k + 1] == vio).astype(BF16)
    t = jnp.dot(oh, wemb_ref[...], preferred_element_type=F32)
    t = jnp.maximum(_mm(t, wb1_ref) + bb1_ref[...], 0.0)
    x = _mm(t, wb2_ref) + bb2_ref[...]
    out_ref[0] = x[:, :128]
    out_ref[1] = x[:, 128:]


def _k2_body(h_ref, wg1_ref, bg1_ref, wg2_ref, bg2_ref, wg3_ref, bg3_ref,
             lng_ref, lnb_ref, out_ref):
    xn = _gin_mlp(h_ref, wg1_ref, bg1_ref, wg2_ref, bg2_ref, wg3_ref,
                  bg3_ref, lng_ref, lnb_ref)
    out_ref[0] = xn[:, :128]
    out_ref[1] = xn[:, 128:]


def _gin_mlp(h_ref, wg1_ref, bg1_ref, wg2_ref, bg2_ref, wg3_ref, bg3_ref,
             lng_ref, lnb_ref):
    h = jnp.concatenate([h_ref[0], h_ref[1]], axis=1)
    h = jnp.maximum(_mm(h, wg1_ref) + bg1_ref[...], 0.0)
    h = jnp.maximum(_mm(h, wg2_ref) + bg2_ref[...], 0.0)
    h = _mm(h, wg3_ref) + bg3_ref[...]
    mu = jnp.mean(h, axis=1, keepdims=True)
    d = h - mu
    var = jnp.mean(d * d, axis=1, keepdims=True)
    return lng_ref[...] * (d * lax.rsqrt(var + 1e-5)) + lnb_ref[...]


def _k3_body(nmol, x1_ref, x2_ref, h3_ref, wg1_ref, bg1_ref, wg2_ref,
             bg2_ref, wg3_ref, bg3_ref, lng_ref, lnb_ref, aux_ref,
             wa1_ref, ba1_ref, wa2_ref, ba2_ref, out_ref):
    i = pl.program_id(0)
    x3 = _gin_mlp(h3_ref, wg1_ref, bg1_ref, wg2_ref, bg2_ref, wg3_ref,
                  bg3_ref, lng_ref, lnb_ref)
    cat = jnp.concatenate([x1_ref[0], x1_ref[1], x2_ref[0], x2_ref[1],
                           x3], axis=1)
    y = jnp.maximum(_mm(cat, wa1_ref) + ba1_ref[...], 0.0)
    z = _mm(y, wa2_ref) + ba2_ref[...]
    sel = (aux_ref[:, 4:5] == lax.broadcasted_iota(I32, (BN, nmol), 1)).astype(BF16)
    contrib = lax.dot_general(sel, z.astype(BF16), (((0,), (0,)), ((), ())),
                              preferred_element_type=F32)

    @pl.when(i == 0)
    def _():
        out_ref[...] = jnp.zeros_like(out_ref)

    out_ref[...] += contrib


def kernel(x_atom_type, x_degree, x_charge, x_hybridization, edge_index,
           batch, ptr, emb_atom, emb_deg, emb_chg, emb_hyb, Wb1, bb1, Wb2,
           bb2, Wg1, bg1, Wg2, bg2, Wg3, bg3, ln_g, ln_b, Wa1, ba1, Wa2, ba2):
    n = x_atom_type.shape[0]
    e = edge_index.shape[1]
    nmol = ptr.shape[0] - 1
    emb = emb_atom.shape[1]
    dim = Wb2.shape[1]
    out_dim = Wa2.shape[1]
    pad_n = N_PAD - n

    o1 = emb_atom.shape[0]
    o2 = o1 + emb_deg.shape[0]
    o3 = o2 + emb_chg.shape[0]
    vocab = o3 + emb_hyb.shape[0]
    zcol = jnp.zeros((n,), I32)
    cols = jnp.stack([x_atom_type.astype(I32), x_degree.astype(I32) + o1,
                      x_charge.astype(I32) + o2,
                      x_hybridization.astype(I32) + o3,
                      batch.astype(I32), zcol, zcol, zcol], axis=1)
    tail = jnp.broadcast_to(
        jnp.array([vocab, vocab, vocab, vocab, nmol, 0, 0, 0], I32),
        (pad_n, 8))
    aux = jnp.concatenate([cols, tail], axis=0)

    w_emb = jnp.concatenate([
        jnp.pad(emb_atom, ((0, 0), (0, 3 * emb))),
        jnp.pad(emb_deg, ((0, 0), (emb, 2 * emb))),
        jnp.pad(emb_chg, ((0, 0), (2 * emb, emb))),
        jnp.pad(emb_hyb, ((0, 0), (3 * emb, 0))),
        jnp.zeros((128 - vocab, 4 * emb), F32)], axis=0)

    src = edge_index[0].astype(I32)
    dst = edge_index[1].astype(I32)
    e_pad = -(-e // (NSUB * EG * 4)) * (NSUB * EG * 4)
    n_groups = e_pad // (NSUB * EG)
    pad_e = e_pad - e
    pad_ar = jnp.arange(pad_e, dtype=I32)
    n_chunk_g = n_groups // 2
    src_g = jnp.concatenate([src, pad_ar % n]).reshape(NSUB * n_groups, EG)
    dst_p = jnp.concatenate([dst, n + pad_ar % pad_n]).reshape(NSUB * n_groups, EG)
    src2 = jnp.concatenate([src_g, src_g + N_PAD], axis=0)

    b1 = bb1.reshape(1, -1)
    b2 = bb2.reshape(1, -1)
    g1 = bg1.reshape(1, -1)
    g2 = bg2.reshape(1, -1)
    g3 = bg3.reshape(1, -1)
    a1 = ba1.reshape(1, -1)
    a2 = ba2.reshape(1, -1)
    lng = ln_g.reshape(1, -1)
    lnb = ln_b.reshape(1, -1)

    xspec = pl.BlockSpec((2, BN, 128), lambda i: (0, i, 0))
    xshape = jax.ShapeDtypeStruct((2, N_PAD, 128), F32)

    x = pl.pallas_call(
        _k1_body,
        grid=(NT,),
        in_specs=[
            pl.BlockSpec((BN, 8), lambda i: (i, 0)),
            _full((128, 4 * emb)),
            _full(Wb1.shape), _full((1, Wb1.shape[1])),
            _full(Wb2.shape), _full((1, dim)),
        ],
        out_specs=xspec,
        out_shape=xshape,
    )(aux, w_emb.astype(BF16), Wb1.astype(BF16), b1, Wb2.astype(BF16), b2)

    gin = pl.pallas_call(
        _k2_body,
        grid=(NT,),
        in_specs=[
            xspec,
            _full(Wg1.shape), _full((1, dim)),
            _full(Wg2.shape), _full((1, dim)),
            _full(Wg3.shape), _full((1, dim)),
            _full((1, dim)), _full((1, dim)),
        ],
        out_specs=xspec,
        out_shape=xshape,
    )
    Wg1b, Wg2b, Wg3b = Wg1.astype(BF16), Wg2.astype(BF16), Wg3.astype(BF16)
    outs = []
    for _ in range(2):
        h2 = _sc_pass(x.reshape(2 * N_PAD, 128), src2, dst_p, n_groups)
        x = gin(h2.reshape(2, N_PAD, 128), Wg1b, g1, Wg2b, g2, Wg3b, g3, lng, lnb)
        outs.append(x)
    h3 = _sc_pass(x.reshape(2 * N_PAD, 128), src2, dst_p, n_groups)
    h3 = h3.reshape(2, N_PAD, 128)

    out = pl.pallas_call(
        functools.partial(_k3_body, nmol),
        grid=(NT,),
        in_specs=[
            xspec, xspec, xspec,
            _full(Wg1.shape), _full((1, dim)),
            _full(Wg2.shape), _full((1, dim)),
            _full(Wg3.shape), _full((1, dim)),
            _full((1, dim)), _full((1, dim)),
            pl.BlockSpec((BN, 8), lambda i: (i, 0)),
            _full(Wa1.shape), _full((1, Wa1.shape[1])),
            _full(Wa2.shape), _full((1, out_dim)),
        ],
        out_specs=pl.BlockSpec((nmol, out_dim), lambda i: (0, 0)),
        out_shape=jax.ShapeDtypeStruct((nmol, out_dim), F32),
    )(outs[0], outs[1], h3, Wg1b, g1, Wg2b, g2, Wg3b, g3, lng, lnb,
      aux, Wa1.astype(BF16), a1, Wa2.astype(BF16), a2)
    return out
